# baseline (speedup 1.0000x reference)
import functools

import jax
import jax.numpy as jnp
from jax import lax
from jax.experimental import pallas as pl
from jax.experimental.pallas import tpu as pltpu
from jax.experimental.pallas import tpu_sc as plsc

B = 1024
D = 64
C = 100000
MOM = 0.5

BC = 1024
NSTEP = pl.cdiv(C, BC)
TOFF = (NSTEP - 1) * BC
TAIL = C - TOFF
W640 = 640
WLAST = 128
NBUF = 4
NSUB = 4
SUBW = BC // NSUB

NC = 2
NS = 16
NW = NC * NS
BPW = B // NW


@functools.cache
def _sc_kernels():
    mesh = plsc.VectorSubcoreMesh(core_axis_name="c", subcore_axis_name="s")
    scratch = [
        pltpu.VMEM((BPW,), jnp.int32),
        pltpu.VMEM((BPW, D), jnp.float32),
        pltpu.SemaphoreType.DMA,
    ]

    CHUNK = 8

    def _row_dmas(hbm, idx_v, rows_v, sem, to_hbm):
        for g in range(BPW // 16):
            vec = idx_v[pl.ds(g * 16, 16)]
            for chunk in range(16 // CHUNK):
                descs = []
                for j in range(CHUNK):
                    lane = chunk * CHUNK + j
                    i = g * 16 + lane
                    c = vec[lane]
                    src = rows_v.at[pl.ds(i, 1)] if to_hbm else hbm.at[pl.ds(c, 1)]
                    dst = hbm.at[pl.ds(c, 1)] if to_hbm else rows_v.at[pl.ds(i, 1)]
                    descs.append(pltpu.async_copy(src, dst, sem))
                for d in descs:
                    d.wait()

    @functools.partial(
        pl.kernel,
        out_type=jax.ShapeDtypeStruct((B, D), jnp.float32),
        mesh=mesh,
        scratch_types=scratch,
    )
    def sc_gather(mem_hbm, y_hbm, out_hbm, idx_v, rows_v, sem):
        wid = lax.axis_index("s") * NC + lax.axis_index("c")
        base = wid * BPW
        pltpu.sync_copy(y_hbm.at[pl.ds(base, BPW)], idx_v)
        _row_dmas(mem_hbm, idx_v, rows_v, sem, False)
        pltpu.sync_copy(rows_v, out_hbm.at[pl.ds(base, BPW)])

    @functools.partial(pl.kernel, out_type=(), mesh=mesh, scratch_types=scratch)
    def sc_scatter(mem_ref, y_hbm, upd_hbm, idx_v, rows_v, sem):
        wid = lax.axis_index("s") * NC + lax.axis_index("c")
        base = wid * BPW
        pltpu.sync_copy(y_hbm.at[pl.ds(base, BPW)], idx_v)
        pltpu.sync_copy(upd_hbm.at[pl.ds(base, BPW)], rows_v)
        _row_dmas(mem_ref, idx_v, rows_v, sem, True)

    return sc_gather, sc_scatter


def _prep_body(f_ref, yc_ref, yr_ref, rows_ref, fn_ref, upd_ref):
    f = f_ref[...]
    fn = f / jnp.sqrt(jnp.sum(f * f, axis=1, keepdims=True))
    fn_ref[...] = fn
    m = (yc_ref[...] == yr_ref[...]).astype(jnp.float32)
    sums = lax.dot_general(
        m, fn, (((1,), (0,)), ((), ())),
        preferred_element_type=jnp.float32,
        precision=lax.Precision.HIGHEST,
    )
    counts = jnp.sum(m, axis=1, keepdims=True)
    mean = sums / counts
    upd = MOM * rows_ref[...] + (1.0 - MOM) * mean
    upd_ref[...] = upd / jnp.sqrt(jnp.sum(upd * upd, axis=1, keepdims=True))


_tc_prep = pl.pallas_call(
    _prep_body,
    out_shape=(
        jax.ShapeDtypeStruct((B, D), jnp.float32),
        jax.ShapeDtypeStruct((B, D), jnp.float32),
    ),
)


def _main_body(fn_ref, mem_hbm, out_hbm, copy_hbm, t32_ref,
               mbuf, obuf, msem, osem, csem):
    i = pl.program_id(0)
    slot = lax.rem(i, NBUF)

    def read_full(j, s):
        return pltpu.make_async_copy(
            mem_hbm.at[pl.ds(j * BC, BC), :], mbuf.at[s], msem.at[s])

    def read_tail(s):
        return pltpu.make_async_copy(
            mem_hbm.at[pl.ds(TOFF, TAIL), :], mbuf.at[s, pl.ds(0, TAIL), :],
            msem.at[s])

    def out_full(j, s):
        return [pltpu.make_async_copy(
            obuf.at[s, :, pl.ds(k * SUBW, SUBW)],
            out_hbm.at[:, pl.ds(j * BC + k * SUBW, SUBW)],
            osem.at[s]) for k in range(NSUB)]

    def out_tail(s):
        return [pltpu.make_async_copy(
            obuf.at[s, :, pl.ds(o, w)],
            out_hbm.at[:, pl.ds(TOFF + o, w)],
            osem.at[s]) for o, w in ((0, 256), (256, 256), (512, 128))]

    def cpy_full(j, s):
        return pltpu.make_async_copy(
            mbuf.at[s], copy_hbm.at[pl.ds(j * BC, BC), :], csem.at[s])

    def cpy_tail(s):
        return pltpu.make_async_copy(
            mbuf.at[s, pl.ds(0, TAIL), :], copy_hbm.at[pl.ds(TOFF, TAIL), :],
            csem.at[s])

    @pl.when(i == 0)
    def _():
        for k in range(NBUF):
            read_full(k, k).start()

    j = i + NBUF - 1
    s2 = lax.rem(j, NBUF)

    @pl.when((i >= 1) & (j <= NSTEP - 2))
    def _():
        cpy_full((i - 1), s2).wait()
        read_full(j, s2).start()

    @pl.when(j == NSTEP - 1)
    def _():
        cpy_full((i - 1), s2).wait()
        read_tail(s2).start()

    @pl.when(i <= NSTEP - 2)
    def _():
        read_full(i, slot).wait()

    @pl.when(i == NSTEP - 1)
    def _():
        read_tail(slot).wait()

    @pl.when(i >= NBUF)
    def _():
        for d in out_full(i - NBUF, slot):
            d.wait()

    res = lax.dot_general(
        fn_ref[...], mbuf[slot], (((1,), (1,)), ((), ())),
        preferred_element_type=jnp.float32,
    )
    obuf[slot] = res

    @pl.when(i == NSTEP - 1)
    def _():
        t32_ref[...] = res[:, W640:W640 + WLAST]

    @pl.when(i <= NSTEP - 2)
    def _():
        for d in out_full(i, slot):
            d.start()
        cpy_full(i, slot).start()

    @pl.when(i == NSTEP - 1)
    def _():
        for d in out_tail(slot):
            d.start()
        cpy_tail(slot).start()
        for k in range(1, NBUF):
            st = NSTEP - 1 - k
            sl = st % NBUF
            for d in out_full(st, sl):
                d.wait()
            cpy_full(st, sl).wait()
        sl = (NSTEP - 1) % NBUF
        for d in out_tail(sl):
            d.wait()
        cpy_tail(sl).wait()


_tc_main = pl.pallas_call(
    _main_body,
    grid=(NSTEP,),
    in_specs=[
        pl.BlockSpec((B, D), lambda i: (0, 0)),
        pl.BlockSpec(memory_space=pl.ANY),
    ],
    out_specs=(
        pl.BlockSpec(memory_space=pl.ANY),
        pl.BlockSpec(memory_space=pl.ANY),
        pl.BlockSpec((B, WLAST), lambda i: (0, 0)),
    ),
    out_shape=(
        jax.ShapeDtypeStruct((B, C), jnp.float32),
        jax.ShapeDtypeStruct((C, D), jnp.float32),
        jax.ShapeDtypeStruct((B, WLAST), jnp.float32),
    ),
    scratch_shapes=[
        pltpu.VMEM((NBUF, BC, D), jnp.float32),
        pltpu.VMEM((NBUF, B, BC), jnp.float32),
        pltpu.SemaphoreType.DMA((NBUF,)),
        pltpu.SemaphoreType.DMA((NBUF,)),
        pltpu.SemaphoreType.DMA((NBUF,)),
    ],
    compiler_params=pltpu.CompilerParams(
        dimension_semantics=("arbitrary",),
    ),
)


def _merge_body(outf_any, t32_ref, out_blk):
    out_blk[...] = t32_ref[...]


_merge32 = pl.pallas_call(
    _merge_body,
    grid=(1,),
    in_specs=[
        pl.BlockSpec(memory_space=pl.ANY),
        pl.BlockSpec((B, WLAST), lambda i: (0, 0)),
    ],
    out_specs=pl.BlockSpec((B, WLAST), lambda i: (0, (TOFF + W640) // WLAST)),
    out_shape=jax.ShapeDtypeStruct((B, C), jnp.float32),
    input_output_aliases={0: 0},
)


def kernel(f, y, memory):
    sc_gather, sc_scatter = _sc_kernels()
    rows = sc_gather(memory, y)
    fn, upd = _tc_prep(f, y.reshape(B, 1), y.reshape(1, B), rows)
    out_main, new_mem, t32 = _tc_main(fn, memory)
    out_f = _merge32(out_main, t32)
    mem_ref = jax.new_ref(new_mem)
    sc_scatter(mem_ref, y, upd)
    return out_f, jax.freeze(mem_ref)


# speedup vs baseline: 1.1980x; 1.1980x over previous
import functools

import jax
import jax.numpy as jnp
from jax import lax
from jax.experimental import pallas as pl
from jax.experimental.pallas import tpu as pltpu
from jax.experimental.pallas import tpu_sc as plsc

B = 1024
D = 64
C = 100000
MOM = 0.5

BN = 8192
BM = 256
NN = pl.cdiv(C, BN)
NM = B // BM

NC = 2
NS = 16
NW = NC * NS
BPW = B // NW


@functools.cache
def _sc_kernels():
    mesh = plsc.VectorSubcoreMesh(core_axis_name="c", subcore_axis_name="s")
    scratch = [
        pltpu.VMEM((BPW,), jnp.int32),
        pltpu.VMEM((BPW, D), jnp.float32),
        pltpu.SemaphoreType.DMA,
    ]

    CHUNK = 8

    def _row_dmas(hbm, idx_v, rows_v, sem, to_hbm):
        for g in range(BPW // 16):
            vec = idx_v[pl.ds(g * 16, 16)]
            for chunk in range(16 // CHUNK):
                descs = []
                for j in range(CHUNK):
                    lane = chunk * CHUNK + j
                    i = g * 16 + lane
                    c = vec[lane]
                    src = rows_v.at[pl.ds(i, 1)] if to_hbm else hbm.at[pl.ds(c, 1)]
                    dst = hbm.at[pl.ds(c, 1)] if to_hbm else rows_v.at[pl.ds(i, 1)]
                    descs.append(pltpu.async_copy(src, dst, sem))
                for d in descs:
                    d.wait()

    @functools.partial(
        pl.kernel,
        out_type=jax.ShapeDtypeStruct((B, D), jnp.float32),
        mesh=mesh,
        scratch_types=scratch,
    )
    def sc_gather(mem_hbm, y_hbm, out_hbm, idx_v, rows_v, sem):
        wid = lax.axis_index("s") * NC + lax.axis_index("c")
        base = wid * BPW
        pltpu.sync_copy(y_hbm.at[pl.ds(base, BPW)], idx_v)
        _row_dmas(mem_hbm, idx_v, rows_v, sem, False)
        pltpu.sync_copy(rows_v, out_hbm.at[pl.ds(base, BPW)])

    @functools.partial(pl.kernel, out_type=(), mesh=mesh, scratch_types=scratch)
    def sc_scatter(mem_ref, y_hbm, upd_hbm, idx_v, rows_v, sem):
        wid = lax.axis_index("s") * NC + lax.axis_index("c")
        base = wid * BPW
        pltpu.sync_copy(y_hbm.at[pl.ds(base, BPW)], idx_v)
        pltpu.sync_copy(upd_hbm.at[pl.ds(base, BPW)], rows_v)
        _row_dmas(mem_ref, idx_v, rows_v, sem, True)

    return sc_gather, sc_scatter


def _prep_body(f_ref, yc_ref, yr_ref, rows_ref, fn_ref, upd_ref):
    f = f_ref[...]
    fn = f / jnp.sqrt(jnp.sum(f * f, axis=1, keepdims=True))
    fn_ref[...] = fn
    m = (yc_ref[...] == yr_ref[...]).astype(jnp.float32)
    sums = lax.dot_general(
        m, fn, (((1,), (0,)), ((), ())),
        preferred_element_type=jnp.float32,
        precision=lax.Precision.HIGHEST,
    )
    counts = jnp.sum(m, axis=1, keepdims=True)
    mean = sums / counts
    upd = MOM * rows_ref[...] + (1.0 - MOM) * mean
    upd_ref[...] = upd / jnp.sqrt(jnp.sum(upd * upd, axis=1, keepdims=True))


_tc_prep = pl.pallas_call(
    _prep_body,
    out_shape=(
        jax.ShapeDtypeStruct((B, D), jnp.float32),
        jax.ShapeDtypeStruct((B, D), jnp.float32),
    ),
)


def _main_body(fn_ref, mem_ref, out_ref, copy_ref):
    mem = mem_ref[...]
    out_ref[...] = lax.dot_general(
        fn_ref[...], mem, (((1,), (1,)), ((), ())),
        preferred_element_type=jnp.float32,
    )
    copy_ref[...] = mem


_tc_main = pl.pallas_call(
    _main_body,
    grid=(NN, NM),
    in_specs=[
        pl.BlockSpec((BM, D), lambda n, m: (m, 0)),
        pl.BlockSpec((BN, D), lambda n, m: (n, 0)),
    ],
    out_specs=(
        pl.BlockSpec((BM, BN), lambda n, m: (m, n)),
        pl.BlockSpec((BN, D), lambda n, m: (n, 0)),
    ),
    out_shape=(
        jax.ShapeDtypeStruct((B, C), jnp.float32),
        jax.ShapeDtypeStruct((C, D), jnp.float32),
    ),
    compiler_params=pltpu.CompilerParams(
        dimension_semantics=("arbitrary", "arbitrary"),
    ),
)


def kernel(f, y, memory):
    sc_gather, sc_scatter = _sc_kernels()
    rows = sc_gather(memory, y)
    fn, upd = _tc_prep(f, y.reshape(B, 1), y.reshape(1, B), rows)
    out_f, new_mem = _tc_main(fn, memory)
    mem_ref = jax.new_ref(new_mem)
    sc_scatter(mem_ref, y, upd)
    return out_f, jax.freeze(mem_ref)
